# Initial kernel scaffold; baseline (speedup 1.0000x reference)
import functools

import jax
import jax.numpy as jnp
from jax import lax
from jax.experimental import pallas as pl
from jax.experimental.pallas import tpu as pltpu
from jax.experimental.pallas import tpu_sc as plsc

_NC = 2
_NS = 16
_NW = _NC * _NS

_CHUNK = 128


def _embed_body(n_chunk, table_hbm, idx_hbm, out_hbm, idx_v, rows_v, sem):
    wid = lax.axis_index("s") * _NC + lax.axis_index("c")
    pltpu.sync_copy(idx_hbm.at[wid], idx_v)

    def body(j, carry):
        pltpu.async_copy(table_hbm.at[idx_v.at[j]], rows_v, sem).wait()
        pltpu.sync_copy(rows_v, out_hbm.at[wid, pl.ds(j * _CHUNK, _CHUNK)])
        return carry

    lax.fori_loop(0, n_chunk, body, 0)


@functools.partial(jax.jit, static_argnums=(2, 3))
def _embed(idx, W, total, d):
    per_w = total // _NW
    n_chunk = per_w // _CHUNK
    mesh = plsc.VectorSubcoreMesh(core_axis_name="c", subcore_axis_name="s")
    k = pl.kernel(
        functools.partial(_embed_body, n_chunk),
        out_type=jax.ShapeDtypeStruct((_NW, per_w, d), jnp.float32),
        mesh=mesh,
        scratch_types=[
            pltpu.VMEM((n_chunk, _CHUNK), jnp.int32),
            pltpu.VMEM((_CHUNK, d), jnp.float32),
            pltpu.SemaphoreType.DMA,
        ],
    )
    out = k(W, idx.reshape(_NW, n_chunk, _CHUNK))
    return out.reshape(total, d)


def kernel(x, W):
    batch, seq = x.shape
    d = W.shape[1]
    total = batch * seq
    idx = x.reshape(-1).astype(jnp.int32)
    out = _embed(idx, W, total, d)
    return out.reshape(batch, seq, d)


# speedup vs baseline: 1.3091x; 1.3091x over previous
import functools

import jax
import jax.numpy as jnp
from jax import lax
from jax.experimental import pallas as pl
from jax.experimental.pallas import tpu as pltpu
from jax.experimental.pallas import tpu_sc as plsc

_NC = 2
_NS = 16
_NW = _NC * _NS

_CHUNK = 128


def _embed_body(n_chunk, table_hbm, idx_hbm, out_hbm, idx_v, rows_v, sem):
    wid = lax.axis_index("s") * _NC + lax.axis_index("c")
    pltpu.sync_copy(idx_hbm.at[wid], idx_v)

    def body(j, carry):
        pltpu.async_copy(table_hbm.at[idx_v.at[j]], rows_v, sem).wait()
        pltpu.sync_copy(rows_v, out_hbm.at[wid, pl.ds(j * _CHUNK, _CHUNK)])
        return carry

    lax.fori_loop(0, n_chunk, body, 0)


@functools.partial(jax.jit, static_argnums=(2, 3))
def _embed(idx, W, total, d):
    per_w = total // _NW
    n_chunk = per_w // _CHUNK
    mesh = plsc.VectorSubcoreMesh(core_axis_name="c", subcore_axis_name="s")
    k = pl.kernel(
        functools.partial(_embed_body, n_chunk),
        out_type=jax.ShapeDtypeStruct((_NW, per_w, d), jnp.float32),
        mesh=mesh,
        scratch_types=[
            pltpu.VMEM((n_chunk, _CHUNK), jnp.int32),
            pltpu.VMEM((_CHUNK, d), jnp.float32),
            pltpu.SemaphoreType.DMA,
        ],
        compiler_params=pltpu.CompilerParams(use_tc_tiling_on_sc=False),
    )
    out = k(W, idx.reshape(_NW, n_chunk, _CHUNK))
    return out.reshape(total, d)


def kernel(x, W):
    batch, seq = x.shape
    d = W.shape[1]
    total = batch * seq
    idx = x.reshape(-1).astype(jnp.int32)
    out = _embed(idx, W, total, d)
    return out.reshape(batch, seq, d)


# speedup vs baseline: 1.4958x; 1.1426x over previous
import functools

import jax
import jax.numpy as jnp
from jax import lax
from jax.experimental import pallas as pl
from jax.experimental.pallas import tpu as pltpu
from jax.experimental.pallas import tpu_sc as plsc

_NC = 2
_NS = 16
_NW = _NC * _NS

_CHUNK = 128


_K = 10
_STEP = _K * _CHUNK


def _embed_body(n_chunk, table_hbm, idx_hbm, out_hbm, idx_v, rows_v, gsem, osem):
    wid = lax.axis_index("s") * _NC + lax.axis_index("c")
    d = table_hbm.shape[1]
    n_step = n_chunk // _K
    pltpu.sync_copy(idx_hbm.at[wid], idx_v)

    def out_drain(b):
        pltpu.make_async_copy(
            rows_v.at[b], out_hbm.at[wid, pl.ds(0, _STEP)], osem
        ).wait()

    def body(i, carry):
        for b in range(2):
            s = i * 2 + b

            @pl.when(s >= 2)
            def _():
                out_drain(b)

            descs = [
                pltpu.make_async_copy(
                    table_hbm.at[idx_v.at[s * _K + k]],
                    rows_v.at[b].at[pl.ds(k * _CHUNK, _CHUNK)],
                    gsem,
                )
                for k in range(_K)
            ]
            for dsc in descs:
                dsc.start()
            for dsc in descs:
                dsc.wait()

            pltpu.make_async_copy(
                rows_v.at[b], out_hbm.at[wid, pl.ds(s * _STEP, _STEP)], osem
            ).start()
        return carry

    lax.fori_loop(0, n_step // 2, body, 0)
    for b in range(2):
        out_drain(b)


@functools.partial(jax.jit, static_argnums=(2, 3))
def _embed(idx, W, total, d):
    per_w = total // _NW
    n_chunk = per_w // _CHUNK
    mesh = plsc.VectorSubcoreMesh(core_axis_name="c", subcore_axis_name="s")
    k = pl.kernel(
        functools.partial(_embed_body, n_chunk),
        out_type=jax.ShapeDtypeStruct((_NW, per_w, d), jnp.float32),
        mesh=mesh,
        scratch_types=[
            pltpu.VMEM((n_chunk, _CHUNK), jnp.int32),
            pltpu.VMEM((2, _STEP, d), jnp.float32),
            pltpu.SemaphoreType.DMA,
            pltpu.SemaphoreType.DMA,
        ],
        compiler_params=pltpu.CompilerParams(use_tc_tiling_on_sc=False),
    )
    out = k(W, idx.reshape(_NW, n_chunk, _CHUNK))
    return out.reshape(total, d)


def kernel(x, W):
    batch, seq = x.shape
    d = W.shape[1]
    total = batch * seq
    idx = x.reshape(-1).astype(jnp.int32)
    out = _embed(idx, W, total, d)
    return out.reshape(batch, seq, d)
